# baseline (speedup 1.0000x reference)
import jax
import jax.numpy as jnp
from jax.experimental import pallas as pl
from jax.experimental.pallas import tpu as pltpu

_B = 4
_N = 256
_NT = _B * _N
_HID = 128
_HEADS = 4
_DH = _HID // _HEADS
_LAYERS = 3
_INV_SQRT_DH = 1.0 / (_DH ** 0.5)


def _fwd(adj_ref, sn_ref, inW_ref, inb_ref, Wq_ref, bq_ref, Wk_ref, bk_ref,
         Wv_ref, bv_ref, Wskip_ref, bskip_ref, ffnW_ref, ffnb_ref,
         lng_ref, lnb_ref, out_ref, h_scr, neg_scr):
    f32 = jnp.float32
    l = pl.program_id(0)
    ones_col = jnp.ones((_N, 1), dtype=f32)

    def _masked_exp(S, neg):
        Sm = S + neg
        m = jnp.maximum(jnp.max(Sm, axis=1, keepdims=True), -1e20)
        return jnp.exp(Sm - m)

    def _agg_norm(E, vh):
        vh1 = jnp.concatenate([vh, ones_col], axis=1)
        full = jnp.dot(E, vh1, preferred_element_type=f32)
        rcp = 1.0 / jnp.maximum(full[:, _DH:_DH + 1], 1e-16)
        return full[:, :_DH] * rcp

    def _tail(agg, skip, h_res):
        hc = agg + skip + bskip_ref[0, 0][None, :]
        hc = hc + jax.nn.relu(
            jnp.dot(hc, ffnW_ref[0], preferred_element_type=f32)
            + ffnb_ref[0, 0][None, :])
        h = hc + h_res
        mu = jnp.mean(h, axis=-1, keepdims=True)
        var = jnp.mean((h - mu) ** 2, axis=-1, keepdims=True)
        h = ((h - mu) * jax.lax.rsqrt(var + 1e-5) * lng_ref[0, 0][None, :]
             + lnb_ref[0, 0][None, :])
        return jax.nn.relu(h)

    @pl.when(l == 0)
    def _layer0():
        sn = sn_ref[...]
        inW = inW_ref[...]
        h1 = (sn[:, 0:1] * inW[0:1, :]
              + sn[:, 1:2] * inW[1:2, :]
              + sn[:, 2:3] * inW[2:3, :]
              + inb_ref[...][None, :])
        for b in range(_B):
            neg_scr[b] = jnp.where(adj_ref[b].T != 0, 0.0, -1e30)
        q1 = (jnp.dot(h1, Wq_ref[0], preferred_element_type=f32)
              + bq_ref[0, 0][None, :]) * _INV_SQRT_DH
        k1 = jnp.dot(h1, Wk_ref[0], preferred_element_type=f32) + bk_ref[0, 0][None, :]
        v1 = jnp.dot(h1, Wv_ref[0], preferred_element_type=f32) + bv_ref[0, 0][None, :]
        skip1 = jnp.dot(h1, Wskip_ref[0], preferred_element_type=f32)
        skip = jnp.broadcast_to(skip1[None], (_B, _N, _HID)).reshape(_NT, _HID)
        head_outs = []
        for hh in range(_HEADS):
            qh = q1[:, hh * _DH:(hh + 1) * _DH]
            kh = k1[:, hh * _DH:(hh + 1) * _DH]
            vh = v1[:, hh * _DH:(hh + 1) * _DH]
            S = jax.lax.dot_general(
                qh, kh, (((1,), (1,)), ((), ())),
                preferred_element_type=f32)
            E = jnp.concatenate(
                [_masked_exp(S, neg_scr[b]) for b in range(_B)],
                axis=0)
            head_outs.append(_agg_norm(E, vh))
        agg = jnp.concatenate(head_outs, axis=1)
        h_res = jnp.broadcast_to(h1[None], (_B, _N, _HID)).reshape(_NT, _HID)
        h_scr[...] = _tail(agg, skip, h_res)

    @pl.when(l > 0)
    def _layer():
        h = h_scr[...]
        q = (jnp.dot(h, Wq_ref[0], preferred_element_type=f32)
             + bq_ref[0, 0][None, :]) * _INV_SQRT_DH
        k = jnp.dot(h, Wk_ref[0], preferred_element_type=f32) + bk_ref[0, 0][None, :]
        v = jnp.dot(h, Wv_ref[0], preferred_element_type=f32) + bv_ref[0, 0][None, :]
        outs = {}
        for hh in range(_HEADS):
            for b in range(_B):
                qh = q[b * _N:(b + 1) * _N, hh * _DH:(hh + 1) * _DH]
                kh = k[b * _N:(b + 1) * _N, hh * _DH:(hh + 1) * _DH]
                vh = v[b * _N:(b + 1) * _N, hh * _DH:(hh + 1) * _DH]
                S = jax.lax.dot_general(
                    qh, kh, (((1,), (1,)), ((), ())),
                    preferred_element_type=f32)
                outs[(b, hh)] = _agg_norm(_masked_exp(S, neg_scr[b]), vh)
        agg = jnp.concatenate(
            [jnp.concatenate([outs[(b, hh)] for hh in range(_HEADS)],
                             axis=1) for b in range(_B)],
            axis=0)
        skip = jnp.dot(h, Wskip_ref[0], preferred_element_type=f32)
        h_scr[...] = _tail(agg, skip, h)

    @pl.when(l == _LAYERS - 1)
    def _emit():
        out_ref[...] = h_scr[...].reshape(_B, _N, _HID).mean(axis=1)


def kernel(adjacency_matrices, single_nodes, in_W, in_b, Wq, bq, Wk, bk,
           Wv, bv, Wskip, bskip, ffn_W, ffn_b, ln_g, ln_b):
    r3 = lambda a: a.reshape(_LAYERS, 1, _HID)
    cst = lambda shape: pl.BlockSpec(shape, lambda l: tuple(0 for _ in shape))
    per_l_w = pl.BlockSpec((1, _HID, _HID), lambda l: (l, 0, 0))
    per_l_b = pl.BlockSpec((1, 1, _HID), lambda l: (l, 0, 0))
    return pl.pallas_call(
        _fwd,
        grid=(_LAYERS,),
        in_specs=[
            cst((_B, _N, _N)),
            cst((_N, 3)),
            cst((3, _HID)), cst((_HID,)),
            per_l_w, per_l_b,
            per_l_w, per_l_b,
            per_l_w, per_l_b,
            per_l_w, per_l_b,
            per_l_w, per_l_b,
            per_l_b, per_l_b,
        ],
        out_specs=pl.BlockSpec((_B, _HID), lambda l: (0, 0)),
        out_shape=jax.ShapeDtypeStruct((_B, _HID), jnp.float32),
        scratch_shapes=[
            pltpu.VMEM((_NT, _HID), jnp.float32),
            pltpu.VMEM((_B, _N, _N), jnp.float32),
        ],
        compiler_params=pltpu.CompilerParams(
            dimension_semantics=("arbitrary",)),
    )(adjacency_matrices, single_nodes, in_W, in_b, Wq, r3(bq), Wk, r3(bk),
      Wv, r3(bv), Wskip, r3(bskip), ffn_W, r3(ffn_b), r3(ln_g), r3(ln_b))


# speedup vs baseline: 1.2572x; 1.2572x over previous
import jax
import jax.numpy as jnp
from jax.experimental import pallas as pl

_B = 4
_N = 256
_NT = _B * _N
_HID = 128
_HEADS = 4
_DH = _HID // _HEADS
_LAYERS = 3
_INV_SQRT_DH = 1.0 / (_DH ** 0.5)


def _fwd(adj_ref, sn_ref, inW_ref, inb_ref, Wq_ref, bq_ref, Wk_ref, bk_ref,
         Wv_ref, bv_ref, Wskip_ref, bskip_ref, ffnW_ref, ffnb_ref,
         lng_ref, lnb_ref, out_ref):
    f32 = jnp.float32

    sn = sn_ref[...]
    inW = inW_ref[...]
    h1 = (sn[:, 0:1] * inW[0:1, :]
          + sn[:, 1:2] * inW[1:1 + 1, :]
          + sn[:, 2:3] * inW[2:2 + 1, :]
          + inb_ref[...][None, :])
    h = jnp.broadcast_to(h1[None], (_B, _N, _HID)).reshape(_NT, _HID)

    negs = [jnp.where(adj_ref[b].T != 0, 0.0, -1e30) for b in range(_B)]
    ones_col = jnp.ones((_N, 1), dtype=f32)

    def _masked_exp(S, b):
        Sm = S + negs[b]
        m = jnp.maximum(jnp.max(Sm, axis=1, keepdims=True), -1e20)
        return jnp.exp(Sm - m)

    def _agg_norm(E, vh):
        vh1 = jnp.concatenate([vh, ones_col], axis=1)
        full = jnp.dot(E, vh1, preferred_element_type=f32)
        rcp = 1.0 / jnp.maximum(full[:, _DH:_DH + 1], 1e-16)
        return full[:, :_DH] * rcp

    for l in range(_LAYERS):
        h_res = h
        if l == 0:
            q1 = (jnp.dot(h1, Wq_ref[l], preferred_element_type=f32)
                  + bq_ref[l][None, :]) * _INV_SQRT_DH
            k1 = jnp.dot(h1, Wk_ref[l], preferred_element_type=f32) + bk_ref[l][None, :]
            v1 = jnp.dot(h1, Wv_ref[l], preferred_element_type=f32) + bv_ref[l][None, :]
            skip1 = jnp.dot(h1, Wskip_ref[l], preferred_element_type=f32)
            skip = jnp.broadcast_to(skip1[None], (_B, _N, _HID)).reshape(_NT, _HID)
            head_outs = []
            for hh in range(_HEADS):
                qh = q1[:, hh * _DH:(hh + 1) * _DH]
                kh = k1[:, hh * _DH:(hh + 1) * _DH]
                vh = v1[:, hh * _DH:(hh + 1) * _DH]
                S = jax.lax.dot_general(
                    qh, kh, (((1,), (1,)), ((), ())),
                    preferred_element_type=f32)
                E = jnp.concatenate([_masked_exp(S, b) for b in range(_B)],
                                    axis=0)
                head_outs.append(_agg_norm(E, vh))
            agg = jnp.concatenate(head_outs, axis=1)
        else:
            q = (jnp.dot(h, Wq_ref[l], preferred_element_type=f32)
                 + bq_ref[l][None, :]) * _INV_SQRT_DH
            k = jnp.dot(h, Wk_ref[l], preferred_element_type=f32) + bk_ref[l][None, :]
            v = jnp.dot(h, Wv_ref[l], preferred_element_type=f32) + bv_ref[l][None, :]
            outs = {}
            for hh in range(_HEADS):
                for b in range(_B):
                    qh = q[b * _N:(b + 1) * _N, hh * _DH:(hh + 1) * _DH]
                    kh = k[b * _N:(b + 1) * _N, hh * _DH:(hh + 1) * _DH]
                    vh = v[b * _N:(b + 1) * _N, hh * _DH:(hh + 1) * _DH]
                    S = jax.lax.dot_general(
                        qh, kh, (((1,), (1,)), ((), ())),
                        preferred_element_type=f32)
                    outs[(b, hh)] = _agg_norm(_masked_exp(S, b), vh)
            agg = jnp.concatenate(
                [jnp.concatenate([outs[(b, hh)] for hh in range(_HEADS)],
                                 axis=1) for b in range(_B)],
                axis=0)
            skip = jnp.dot(h, Wskip_ref[l], preferred_element_type=f32)

        hc = agg + skip + bskip_ref[l][None, :]
        hc = hc + jax.nn.relu(
            jnp.dot(hc, ffnW_ref[l], preferred_element_type=f32) + ffnb_ref[l][None, :])
        h = hc + h_res
        mu = jnp.mean(h, axis=-1, keepdims=True)
        var = jnp.mean((h - mu) ** 2, axis=-1, keepdims=True)
        h = (h - mu) * jax.lax.rsqrt(var + 1e-5) * lng_ref[l][None, :] + lnb_ref[l][None, :]
        h = jax.nn.relu(h)

    out_ref[...] = h.reshape(_B, _N, _HID).mean(axis=1)


def kernel(adjacency_matrices, single_nodes, in_W, in_b, Wq, bq, Wk, bk,
           Wv, bv, Wskip, bskip, ffn_W, ffn_b, ln_g, ln_b):
    return pl.pallas_call(
        _fwd,
        out_shape=jax.ShapeDtypeStruct((_B, _HID), jnp.float32),
    )(adjacency_matrices, single_nodes, in_W, in_b, Wq, bq, Wk, bk,
      Wv, bv, Wskip, bskip, ffn_W, ffn_b, ln_g, ln_b)
